# Initial kernel scaffold; baseline (speedup 1.0000x reference)
import jax
import jax.numpy as jnp
from jax.experimental import pallas as pl


def kernel(embeds_stack, ent_embeds, rel_embeds, dt_vals, segment_ids, s_len_non_zero, s_tem, r_tem):
    raise NotImplementedError("write your pallas kernel here")


# speedup vs baseline: 1.5115x; 1.5115x over previous
import functools

import jax
import jax.numpy as jnp
from jax import lax
from jax.experimental import pallas as pl
from jax.experimental.pallas import tpu as pltpu
from jax.experimental.pallas import tpu_sc as plsc

S = 20480
SEQ_LEN = 10
H_DIM = 128
EMBD_RANK = 64
N = 320000
B = 2048
N_ENT = 10000
N_REL = 500

NC = 2
NS = 16

AC = 80
SROWS = 20608
SLAB = SROWS // NS
MBLK = 56
NMB = SLAB // MBLK
RPT = N // NS
RB1 = 80
NRB = RPT // RB1
RB2 = 128
QPT = (B * SEQ_LEN) // NS
NQB = QPT // RB2


def _body(emb, sids, dt_ext, ent_ext, rel_ext, idxf, eidx, ridx, onespad,
          zer, full, dtout, acc, rbuf, idxb1, idxb, mbuf, dtb, ebuf, eixb,
          sem):
    cid = lax.axis_index("c")
    sid = lax.axis_index("s")

    pltpu.sync_copy(zer, mbuf)
    for b in range(NMB):
        pltpu.sync_copy(mbuf, acc.at[pl.ds(sid * SLAB + b * MBLK, MBLK), :])
    pltpu.sync_copy(onespad, rbuf.at[:, 64:AC])
    plsc.subcore_barrier()

    col0 = cid * 64

    def p1(i, _):
        r0 = sid * RPT + i * RB1
        pltpu.sync_copy(emb.at[pl.ds(r0, RB1), pl.ds(col0, 64)],
                        rbuf.at[pl.ds(0, RB1), 0:64])
        pltpu.sync_copy(sids.at[pl.ds(r0, RB1)], idxb1)
        pltpu.sync_copy(rbuf.at[pl.ds(0, RB1), :], acc.at[idxb1], add=True)
        return 0

    lax.fori_loop(0, NRB, p1, 0)
    plsc.subcore_barrier()

    def scale_row(r, _):
        v = mbuf[r, 64:80]
        rv = 1.0 / jnp.maximum(v, 1.0)
        s = rv[0]
        for j in range(4):
            mbuf[r, j * 16:(j + 1) * 16] = mbuf[r, j * 16:(j + 1) * 16] * s
        mbuf[r, 64:80] = dtb[pl.ds(r, 16)]
        return 0

    for b in range(NMB):
        row0 = sid * SLAB + b * MBLK
        pltpu.sync_copy(acc.at[pl.ds(row0, MBLK), :], mbuf)
        pltpu.sync_copy(dt_ext.at[pl.ds(row0, MBLK)], dtb.at[pl.ds(0, MBLK)])
        lax.fori_loop(0, MBLK, scale_row, 0)
        pltpu.sync_copy(mbuf, acc.at[pl.ds(row0, MBLK), :])
    plsc.subcore_barrier()

    def p2(k, _):
        k0 = sid * QPT + k * RB2
        pltpu.sync_copy(idxf.at[pl.ds(k0, RB2)], idxb)
        pltpu.async_copy(acc.at[idxb], rbuf, sem).wait()

        @pl.when(cid == 0)
        def _():
            pltpu.sync_copy(rbuf.at[:, 0:64], full.at[pl.ds(k0, RB2), 0:64])
            pltpu.sync_copy(rbuf.at[:, 64:65], dtout.at[pl.ds(k0, RB2), :])
            pltpu.sync_copy(eidx.at[pl.ds(k0, RB2)], eixb)
            pltpu.async_copy(ent_ext.at[eixb], ebuf, sem).wait()
            pltpu.sync_copy(ebuf, full.at[pl.ds(k0, RB2), 128:192])

        @pl.when(cid == 1)
        def _():
            pltpu.sync_copy(rbuf.at[:, 0:64], full.at[pl.ds(k0, RB2), 64:128])
            pltpu.sync_copy(ridx.at[pl.ds(k0, RB2)], eixb)
            pltpu.async_copy(rel_ext.at[eixb], ebuf, sem).wait()
            pltpu.sync_copy(ebuf, full.at[pl.ds(k0, RB2), 192:256])

        return 0

    lax.fori_loop(0, NQB, p2, 0)


_sc_call = functools.partial(
    pl.kernel,
    out_type=(
        jax.ShapeDtypeStruct((B * SEQ_LEN, 256), jnp.float32),
        jax.ShapeDtypeStruct((B * SEQ_LEN, 1), jnp.float32),
    ),
    mesh=plsc.VectorSubcoreMesh(core_axis_name="c", subcore_axis_name="s"),
    compiler_params=pltpu.CompilerParams(use_tc_tiling_on_sc=False),
    scratch_types=[
        pltpu.VMEM_SHARED((SROWS, AC), jnp.float32),
        pltpu.VMEM((RB2, AC), jnp.float32),
        pltpu.VMEM((RB1,), jnp.int32),
        pltpu.VMEM((RB2,), jnp.int32),
        pltpu.VMEM((MBLK, AC), jnp.float32),
        pltpu.VMEM((MBLK + 16,), jnp.float32),
        pltpu.VMEM((RB2, EMBD_RANK), jnp.float32),
        pltpu.VMEM((RB2,), jnp.int32),
        pltpu.SemaphoreType.DMA,
    ],
)(_body)


def kernel(embeds_stack, ent_embeds, rel_embeds, dt_vals, segment_ids,
           s_len_non_zero, s_tem, r_tem):
    lens = s_len_non_zero.astype(jnp.int32)
    offsets = jnp.concatenate(
        [jnp.zeros((1,), jnp.int32), jnp.cumsum(lens)[:-1]])
    pos = jnp.arange(SEQ_LEN, dtype=jnp.int32)
    idx = offsets[:, None] + pos[None, :]
    mask = pos[None, :] < lens[:, None]
    idxf = jnp.where(mask, idx, S).reshape(-1).astype(jnp.int32)
    eidx = jnp.where(mask, s_tem.astype(jnp.int32)[:, None],
                     N_ENT).reshape(-1)
    ridx = jnp.where(mask, r_tem.astype(jnp.int32)[:, None],
                     N_REL).reshape(-1)

    dt_ext = jnp.zeros((SROWS,), jnp.float32).at[:S].set(dt_vals)
    ent_ext = jnp.zeros((N_ENT + 8, EMBD_RANK), jnp.float32).at[:N_ENT].set(
        ent_embeds)
    rel_ext = jnp.zeros((N_REL + 8, EMBD_RANK), jnp.float32).at[:N_REL].set(
        rel_embeds)
    onespad = jnp.zeros((RB2, AC - 64), jnp.float32).at[:, 0].set(1.0)
    zer = jnp.zeros((MBLK, AC), jnp.float32)

    full, dt = _sc_call(embeds_stack, segment_ids.astype(jnp.int32), dt_ext,
                        ent_ext, rel_ext, idxf, eidx, ridx, onespad, zer)
    return full.reshape(B, SEQ_LEN, 256), dt.reshape(B, SEQ_LEN)


# speedup vs baseline: 2.1568x; 1.4270x over previous
import functools

import jax
import jax.numpy as jnp
from jax import lax
from jax.experimental import pallas as pl
from jax.experimental.pallas import tpu as pltpu
from jax.experimental.pallas import tpu_sc as plsc

S = 20480
SEQ_LEN = 10
H_DIM = 128
EMBD_RANK = 64
N = 320000
B = 2048
N_ENT = 10000
N_REL = 500

NC = 2
NS = 16

AC = 80
SROWS = 20608
SLAB = SROWS // NS
MBLK = 56
NMB = SLAB // MBLK
RPT = N // NS
RB1 = 80
NRB = RPT // RB1
NPAIR = NRB // 2
RB2 = 80
QPT = (B * SEQ_LEN) // NS
NQB = QPT // RB2


def _body(emb, sids, dt_ext, ent_ext, rel_ext, idxf, eidx, ridx, onespad,
          zer, full, dtout, acc, rbufA, rbufB, idxbA, idxbB, mbuf, dtb,
          ebuf, eixb, semL, semAA, semAB):
    cid = lax.axis_index("c")
    sid = lax.axis_index("s")
    col0 = cid * 64

    pltpu.sync_copy(zer, mbuf)
    for b in range(NMB):
        pltpu.sync_copy(mbuf, acc.at[pl.ds(sid * SLAB + b * MBLK, MBLK), :])
    pltpu.sync_copy(onespad, rbufA.at[:, 64:AC])
    pltpu.sync_copy(onespad, rbufB.at[:, 64:AC])
    plsc.subcore_barrier()

    def loads(buf, ib, i):
        r0 = sid * RPT + i * RB1
        return (
            pltpu.make_async_copy(
                emb.at[pl.ds(r0, RB1), pl.ds(col0, 64)], buf.at[:, 0:64],
                semL),
            pltpu.make_async_copy(sids.at[pl.ds(r0, RB1)], ib, semL),
        )

    def start_loads(buf, ib, i):
        for d in loads(buf, ib, i):
            d.start()

    def wait_loads(buf, ib, i):
        for d in loads(buf, ib, i):
            d.wait()

    start_loads(rbufA, idxbA, 0)

    def pair(j, _):
        @pl.when(j > 0)
        def _():
            pltpu.make_async_copy(rbufB, acc.at[idxbB], semAB).wait()

        start_loads(rbufB, idxbB, 2 * j + 1)
        wait_loads(rbufA, idxbA, 2 * j)
        pltpu.async_copy(rbufA, acc.at[idxbA], semAA, add=True)
        pltpu.make_async_copy(rbufA, acc.at[idxbA], semAA).wait()

        @pl.when(j < NPAIR - 1)
        def _():
            start_loads(rbufA, idxbA, 2 * j + 2)

        wait_loads(rbufB, idxbB, 2 * j + 1)
        pltpu.async_copy(rbufB, acc.at[idxbB], semAB, add=True)
        return 0

    lax.fori_loop(0, NPAIR, pair, 0)
    pltpu.make_async_copy(rbufB, acc.at[idxbB], semAB).wait()
    plsc.subcore_barrier()

    def scale_row(r, _):
        v = mbuf[r, 64:80]
        rv = 1.0 / jnp.maximum(v, 1.0)
        s = rv[0]
        for j in range(4):
            mbuf[r, j * 16:(j + 1) * 16] = mbuf[r, j * 16:(j + 1) * 16] * s
        mbuf[r, 64:80] = dtb[pl.ds(r, 16)]
        return 0

    for b in range(NMB):
        row0 = sid * SLAB + b * MBLK
        pltpu.sync_copy(acc.at[pl.ds(row0, MBLK), :], mbuf)
        pltpu.sync_copy(dt_ext.at[pl.ds(row0, MBLK)], dtb.at[pl.ds(0, MBLK)])
        lax.fori_loop(0, MBLK, scale_row, 0)
        pltpu.sync_copy(mbuf, acc.at[pl.ds(row0, MBLK), :])
    plsc.subcore_barrier()

    def p2(k, _):
        k0 = sid * QPT + k * RB2
        pltpu.sync_copy(idxf.at[pl.ds(k0, RB2)], idxbA)
        pltpu.async_copy(acc.at[idxbA], rbufA, semL).wait()

        @pl.when(cid == 0)
        def _():
            pltpu.sync_copy(rbufA.at[:, 0:64], full.at[pl.ds(k0, RB2), 0:64])
            pltpu.sync_copy(rbufA.at[:, 64:65], dtout.at[pl.ds(k0, RB2), :])
            pltpu.sync_copy(eidx.at[pl.ds(k0, RB2)], eixb)
            pltpu.async_copy(ent_ext.at[eixb], ebuf, semL).wait()
            pltpu.sync_copy(ebuf, full.at[pl.ds(k0, RB2), 128:192])

        @pl.when(cid == 1)
        def _():
            pltpu.sync_copy(rbufA.at[:, 0:64], full.at[pl.ds(k0, RB2), 64:128])
            pltpu.sync_copy(ridx.at[pl.ds(k0, RB2)], eixb)
            pltpu.async_copy(rel_ext.at[eixb], ebuf, semL).wait()
            pltpu.sync_copy(ebuf, full.at[pl.ds(k0, RB2), 192:256])

        return 0

    lax.fori_loop(0, NQB, p2, 0)


_sc_call = functools.partial(
    pl.kernel,
    out_type=(
        jax.ShapeDtypeStruct((B * SEQ_LEN, 256), jnp.float32),
        jax.ShapeDtypeStruct((B * SEQ_LEN, 1), jnp.float32),
    ),
    mesh=plsc.VectorSubcoreMesh(core_axis_name="c", subcore_axis_name="s"),
    compiler_params=pltpu.CompilerParams(use_tc_tiling_on_sc=False),
    scratch_types=[
        pltpu.VMEM_SHARED((SROWS, AC), jnp.float32),
        pltpu.VMEM((RB1, AC), jnp.float32),
        pltpu.VMEM((RB1, AC), jnp.float32),
        pltpu.VMEM((RB1,), jnp.int32),
        pltpu.VMEM((RB1,), jnp.int32),
        pltpu.VMEM((MBLK, AC), jnp.float32),
        pltpu.VMEM((MBLK + 16,), jnp.float32),
        pltpu.VMEM((RB2, EMBD_RANK), jnp.float32),
        pltpu.VMEM((RB2,), jnp.int32),
        pltpu.SemaphoreType.DMA,
        pltpu.SemaphoreType.DMA,
        pltpu.SemaphoreType.DMA,
    ],
)(_body)


def kernel(embeds_stack, ent_embeds, rel_embeds, dt_vals, segment_ids,
           s_len_non_zero, s_tem, r_tem):
    lens = s_len_non_zero.astype(jnp.int32)
    offsets = jnp.concatenate(
        [jnp.zeros((1,), jnp.int32), jnp.cumsum(lens)[:-1]])
    pos = jnp.arange(SEQ_LEN, dtype=jnp.int32)
    idx = offsets[:, None] + pos[None, :]
    mask = pos[None, :] < lens[:, None]
    idxf = jnp.where(mask, idx, S).reshape(-1).astype(jnp.int32)
    eidx = jnp.where(mask, s_tem.astype(jnp.int32)[:, None],
                     N_ENT).reshape(-1)
    ridx = jnp.where(mask, r_tem.astype(jnp.int32)[:, None],
                     N_REL).reshape(-1)

    dt_ext = jnp.zeros((SROWS,), jnp.float32).at[:S].set(dt_vals)
    ent_ext = jnp.zeros((N_ENT + 8, EMBD_RANK), jnp.float32).at[:N_ENT].set(
        ent_embeds)
    rel_ext = jnp.zeros((N_REL + 8, EMBD_RANK), jnp.float32).at[:N_REL].set(
        rel_embeds)
    onespad = jnp.zeros((RB1, AC - 64), jnp.float32).at[:, 0].set(1.0)
    zer = jnp.zeros((MBLK, AC), jnp.float32)

    full, dt = _sc_call(embeds_stack, segment_ids.astype(jnp.int32), dt_ext,
                        ent_ext, rel_ext, idxf, eidx, ridx, onespad, zer)
    return full.reshape(B, SEQ_LEN, 256), dt.reshape(B, SEQ_LEN)
